# Initial kernel scaffold; baseline (speedup 1.0000x reference)
import functools

import jax
import jax.numpy as jnp
from jax import lax
from jax.experimental import pallas as pl
from jax.experimental.pallas import tpu as pltpu
from jax.experimental.pallas import tpu_sc as plsc

N = 10000
E = 320000
D = 128
DH = D // 2
NCORE = 2
NSUB = 16
NW = NCORE * NSUB

f32 = jnp.float32


def _sc_mesh():
    return plsc.VectorSubcoreMesh(
        core_axis_name="c", subcore_axis_name="s",
        num_cores=NCORE, num_subcores=NSUB)


_BN = 1000
_BE = 2000


def _node_proj(x, wn, bn):
    def body(x_ref, w_ref, b_ref, ax_ref, dbx_ref, ex_ref):
        r = jnp.dot(x_ref[...], w_ref[...], preferred_element_type=f32) + b_ref[...]
        ax_ref[...] = r[:, :D]
        dbx_ref[...] = r[:, D:3 * D]
        ex_ref[...] = r[:, 3 * D:]

    return pl.pallas_call(
        body, grid=(N // _BN,),
        in_specs=[pl.BlockSpec((_BN, D), lambda i: (i, 0)),
                  pl.BlockSpec((D, 4 * D), lambda i: (0, 0)),
                  pl.BlockSpec((1, 4 * D), lambda i: (0, 0))],
        out_specs=[pl.BlockSpec((_BN, D), lambda i: (i, 0)),
                   pl.BlockSpec((_BN, 2 * D), lambda i: (i, 0)),
                   pl.BlockSpec((_BN, D), lambda i: (i, 0))],
        out_shape=[jax.ShapeDtypeStruct((N, D), f32),
                   jax.ShapeDtypeStruct((N, 2 * D), f32),
                   jax.ShapeDtypeStruct((N, D), f32)],
    )(x, wn, bn.reshape(1, -1))


def _edge_mm(e, w, b):
    K = e.shape[1]

    def body(e_ref, w_ref, b_ref, o_ref):
        o_ref[...] = jnp.dot(e_ref[...], w_ref[...], preferred_element_type=f32) + b_ref[...]

    return pl.pallas_call(
        body, grid=(E // _BE,),
        in_specs=[pl.BlockSpec((_BE, K), lambda i: (i, 0)),
                  pl.BlockSpec((K, D), lambda i: (0, 0)),
                  pl.BlockSpec((1, D), lambda i: (0, 0))],
        out_specs=pl.BlockSpec((_BE, D), lambda i: (i, 0)),
        out_shape=jax.ShapeDtypeStruct((E, D), f32),
    )(e, w, b.reshape(1, -1))


def _edge_elem(ce, dbs, exd, eprev, ea, wep, bep):
    layer0 = eprev is None

    def body(ce_ref, dbs_ref, exd_ref, ep_ref, w_ref, b_ref, msg_ref, sig_ref, eo_ref):
        dxs = dbs_ref[:, :D]
        bxs = dbs_ref[:, D:]
        e_new = dxs + exd_ref[...] + ce_ref[...]
        sg = jax.nn.sigmoid(e_new)
        msg = sg * bxs
        if layer0:
            ep = jnp.dot(ep_ref[...], w_ref[...], preferred_element_type=f32) + b_ref[...]
        else:
            ep = ep_ref[...]
        eo_ref[...] = ep + jnp.maximum(e_new, 0.0)
        msg_ref[...] = jnp.stack([msg[:, :DH], msg[:, DH:]])
        sig_ref[...] = jnp.stack([sg[:, :DH], sg[:, DH:]])

    ep_in = ea if layer0 else eprev
    Kp = ep_in.shape[1]
    return pl.pallas_call(
        body, grid=(E // _BE,),
        in_specs=[pl.BlockSpec((_BE, D), lambda i: (i, 0)),
                  pl.BlockSpec((_BE, 2 * D), lambda i: (i, 0)),
                  pl.BlockSpec((_BE, D), lambda i: (i, 0)),
                  pl.BlockSpec((_BE, Kp), lambda i: (i, 0)),
                  pl.BlockSpec((wep.shape[0], D), lambda i: (0, 0)),
                  pl.BlockSpec((1, D), lambda i: (0, 0))],
        out_specs=[pl.BlockSpec((2, _BE, DH), lambda i: (0, i, 0)),
                   pl.BlockSpec((2, _BE, DH), lambda i: (0, i, 0)),
                   pl.BlockSpec((_BE, D), lambda i: (i, 0))],
        out_shape=[jax.ShapeDtypeStruct((2, E, DH), f32),
                   jax.ShapeDtypeStruct((2, E, DH), f32),
                   jax.ShapeDtypeStruct((E, D), f32)],
    )(ce, dbs, exd, ep_in, wep, bep.reshape(1, -1))


def _x_update(x, ax, num2, den2, wh=None, bh=None):
    head = wh is not None

    def body(x_ref, ax_ref, n_ref, d_ref, w_ref, b_ref, o_ref):
        num = jnp.concatenate([n_ref[0], n_ref[1]], axis=1)
        den = jnp.concatenate([d_ref[0], d_ref[1]], axis=1)
        xn = x_ref[...] + jnp.maximum(ax_ref[...] + num / (den + 1e-6), 0.0)
        if head:
            o_ref[...] = jnp.dot(xn, w_ref[...], preferred_element_type=f32) + b_ref[...]
        else:
            o_ref[...] = xn

    DO = wh.shape[1] if head else D
    if wh is None:
        wh = jnp.zeros((D, D), f32)
        bh = jnp.zeros((D,), f32)
    return pl.pallas_call(
        body, grid=(N // _BN,),
        in_specs=[pl.BlockSpec((_BN, D), lambda i: (i, 0)),
                  pl.BlockSpec((_BN, D), lambda i: (i, 0)),
                  pl.BlockSpec((2, _BN, DH), lambda i: (0, i, 0)),
                  pl.BlockSpec((2, _BN, DH), lambda i: (0, i, 0)),
                  pl.BlockSpec((D, DO), lambda i: (0, 0)),
                  pl.BlockSpec((1, DO), lambda i: (0, 0))],
        out_specs=pl.BlockSpec((_BN, DO), lambda i: (i, 0)),
        out_shape=jax.ShapeDtypeStruct((N, DO), f32),
    )(x, ax, num2, den2, wh, bh.reshape(1, -1))


_EPW = E // NW
_GCH = 80
_GNCH = _EPW // _GCH

_EPT = E // NSUB
_SCH = 80
_SNCH = _EPT // _SCH
_RPT = N // NSUB


def _sc_gather(dbx, ext, src, dst):
    @functools.partial(
        pl.kernel,
        out_type=(jax.ShapeDtypeStruct((E, 2 * D), f32),
                  jax.ShapeDtypeStruct((E, D), f32)),
        mesh=_sc_mesh(),
        scratch_types=[
            pltpu.VMEM((_GCH,), jnp.int32),
            pltpu.VMEM((_GCH,), jnp.int32),
            pltpu.VMEM((_GCH, 2 * D), f32),
            pltpu.VMEM((_GCH, D), f32),
            pltpu.SemaphoreType.DMA,
            pltpu.SemaphoreType.DMA,
        ],
    )
    def k(dbx_hbm, ext_hbm, src_hbm, dst_hbm, dbs_hbm, exd_hbm,
          sbuf, tbuf, dbuf, ebuf, sem1, sem2):
        wid = lax.axis_index("s") * NCORE + lax.axis_index("c")
        base0 = wid * _EPW

        @pl.loop(0, _GNCH)
        def _(ci):
            base = base0 + ci * _GCH
            pltpu.sync_copy(src_hbm.at[pl.ds(base, _GCH)], sbuf)
            pltpu.sync_copy(dst_hbm.at[pl.ds(base, _GCH)], tbuf)
            cp1 = pltpu.async_copy(dbx_hbm.at[sbuf], dbuf, sem1)
            cp2 = pltpu.async_copy(ext_hbm.at[tbuf], ebuf, sem2)
            cp1.wait()
            cp2.wait()
            pltpu.sync_copy(dbuf, dbs_hbm.at[pl.ds(base, _GCH)])
            pltpu.sync_copy(ebuf, exd_hbm.at[pl.ds(base, _GCH)])

    return k(dbx, ext, src, dst)


def _sc_scatter(msg2, sig2, dst):
    @functools.partial(
        pl.kernel,
        out_type=(jax.ShapeDtypeStruct((2, N, DH), f32),
                  jax.ShapeDtypeStruct((2, N, DH), f32)),
        mesh=_sc_mesh(),
        scratch_types=[
            pltpu.VMEM((_SCH,), jnp.int32),
            pltpu.VMEM((_SCH, DH), f32),
            pltpu.VMEM((_SCH, DH), f32),
            pltpu.VMEM((_RPT, DH), f32),
            pltpu.VMEM_SHARED((N, DH), f32),
            pltpu.VMEM_SHARED((N, DH), f32),
        ],
    )
    def k(msg_hbm, sig_hbm, dst_hbm, num_hbm, den_hbm,
          ibuf, mbuf, gbuf, zbuf, num_sh, den_sh):
        cid = lax.axis_index("c")
        sid = lax.axis_index("s")
        row0 = sid * _RPT

        @pl.loop(0, _RPT)
        def _(r):
            for j in range(DH // 16):
                zbuf[r, pl.ds(j * 16, 16)] = jnp.zeros((16,), f32)

        pltpu.sync_copy(zbuf, num_sh.at[pl.ds(row0, _RPT)])
        pltpu.sync_copy(zbuf, den_sh.at[pl.ds(row0, _RPT)])
        plsc.subcore_barrier()

        base0 = sid * _EPT

        @pl.loop(0, _SNCH)
        def _(ci):
            base = base0 + ci * _SCH
            pltpu.sync_copy(dst_hbm.at[pl.ds(base, _SCH)], ibuf)
            pltpu.sync_copy(msg_hbm.at[cid, pl.ds(base, _SCH)], mbuf)
            pltpu.sync_copy(sig_hbm.at[cid, pl.ds(base, _SCH)], gbuf)
            pltpu.sync_copy(mbuf, num_sh.at[ibuf], add=True)
            pltpu.sync_copy(gbuf, den_sh.at[ibuf], add=True)

        plsc.subcore_barrier()
        pltpu.sync_copy(num_sh.at[pl.ds(row0, _RPT)], num_hbm.at[cid, pl.ds(row0, _RPT)])
        pltpu.sync_copy(den_sh.at[pl.ds(row0, _RPT)], den_hbm.at[cid, pl.ds(row0, _RPT)])

    return k(msg2, sig2, dst)


def kernel(x, edge_index, edge_attr, We, be, Ws, bs, Wh, bh):
    src = edge_index[0]
    dst = edge_index[1]
    ea = jnp.pad(edge_attr, ((0, 0), (0, 1)))
    wep = jnp.pad(We, ((0, 1), (0, 0)))

    e = None
    out = None
    for l in range(3):
        W = Ws[l]
        b = bs[l]
        wn = jnp.concatenate([W[0], W[3], W[1], W[4]], axis=1)
        bn = jnp.concatenate([b[0], b[3], b[1], b[4]])
        ax, dbx, ext = _node_proj(x, wn, bn)
        if l == 0:
            ce = _edge_mm(ea, wep @ W[2], be @ W[2] + b[2])
        else:
            ce = _edge_mm(e, W[2], b[2])
        dbs, exd = _sc_gather(dbx, ext, src, dst)
        msg2, sig2, e = _edge_elem(ce, dbs, exd, e, ea, wep, be)
        num2, den2 = _sc_scatter(msg2, sig2, dst)
        if l < 2:
            x = _x_update(x, ax, num2, den2)
        else:
            out = _x_update(x, ax, num2, den2, Wh, bh)
    return out


# speedup vs baseline: 2.5071x; 2.5071x over previous
import functools

import jax
import jax.numpy as jnp
from jax import lax
from jax.experimental import pallas as pl
from jax.experimental.pallas import tpu as pltpu
from jax.experimental.pallas import tpu_sc as plsc

N = 10000
E = 320000
D = 128
DH = D // 2
NCORE = 2
NSUB = 16
NW = NCORE * NSUB

f32 = jnp.float32


def _sc_mesh():
    return plsc.VectorSubcoreMesh(
        core_axis_name="c", subcore_axis_name="s",
        num_cores=NCORE, num_subcores=NSUB)


_BN = 1000
_BE = 2000


def _node_proj(x, wn, bn):
    def body(x_ref, w_ref, b_ref, ax_ref, dbx_ref, ex_ref):
        r = jnp.dot(x_ref[...], w_ref[...], preferred_element_type=f32) + b_ref[...]
        ax_ref[...] = r[:, :D]
        dbx_ref[...] = r[:, D:3 * D]
        ex_ref[...] = r[:, 3 * D:]

    return pl.pallas_call(
        body, grid=(N // _BN,),
        in_specs=[pl.BlockSpec((_BN, D), lambda i: (i, 0)),
                  pl.BlockSpec((D, 4 * D), lambda i: (0, 0)),
                  pl.BlockSpec((1, 4 * D), lambda i: (0, 0))],
        out_specs=[pl.BlockSpec((_BN, D), lambda i: (i, 0)),
                   pl.BlockSpec((_BN, 2 * D), lambda i: (i, 0)),
                   pl.BlockSpec((_BN, D), lambda i: (i, 0))],
        out_shape=[jax.ShapeDtypeStruct((N, D), f32),
                   jax.ShapeDtypeStruct((N, 2 * D), f32),
                   jax.ShapeDtypeStruct((N, D), f32)],
    )(x, wn, bn.reshape(1, -1))


def _edge_mm(e, w, b):
    K = e.shape[1]

    def body(e_ref, w_ref, b_ref, o_ref):
        o_ref[...] = jnp.dot(e_ref[...], w_ref[...], preferred_element_type=f32) + b_ref[...]

    return pl.pallas_call(
        body, grid=(E // _BE,),
        in_specs=[pl.BlockSpec((_BE, K), lambda i: (i, 0)),
                  pl.BlockSpec((K, D), lambda i: (0, 0)),
                  pl.BlockSpec((1, D), lambda i: (0, 0))],
        out_specs=pl.BlockSpec((_BE, D), lambda i: (i, 0)),
        out_shape=jax.ShapeDtypeStruct((E, D), f32),
    )(e, w, b.reshape(1, -1))


def _edge_elem(ce, dbs, exd, eprev, ea, wep, bep):
    layer0 = eprev is None

    def body(ce_ref, dbs_ref, exd_ref, ep_ref, w_ref, b_ref, msg_ref, eo_ref):
        dxs = dbs_ref[:, :D]
        bxs = dbs_ref[:, D:]
        e_new = dxs + exd_ref[...] + ce_ref[...]
        sg = jax.nn.sigmoid(e_new)
        msg = sg * bxs
        if layer0:
            ep = jnp.dot(ep_ref[...], w_ref[...], preferred_element_type=f32) + b_ref[...]
        else:
            ep = ep_ref[...]
        eo_ref[...] = ep + jnp.maximum(e_new, 0.0)
        msg_ref[...] = jnp.stack(
            [jnp.concatenate([msg[:, :DH], sg[:, :DH]], axis=1),
             jnp.concatenate([msg[:, DH:], sg[:, DH:]], axis=1)])

    ep_in = ea if layer0 else eprev
    Kp = ep_in.shape[1]
    return pl.pallas_call(
        body, grid=(E // _BE,),
        in_specs=[pl.BlockSpec((_BE, D), lambda i: (i, 0)),
                  pl.BlockSpec((_BE, 2 * D), lambda i: (i, 0)),
                  pl.BlockSpec((_BE, D), lambda i: (i, 0)),
                  pl.BlockSpec((_BE, Kp), lambda i: (i, 0)),
                  pl.BlockSpec((wep.shape[0], D), lambda i: (0, 0)),
                  pl.BlockSpec((1, D), lambda i: (0, 0))],
        out_specs=[pl.BlockSpec((2, _BE, D), lambda i: (0, i, 0)),
                   pl.BlockSpec((_BE, D), lambda i: (i, 0))],
        out_shape=[jax.ShapeDtypeStruct((2, E, D), f32),
                   jax.ShapeDtypeStruct((E, D), f32)],
    )(ce, dbs, exd, ep_in, wep, bep.reshape(1, -1))


def _x_update(x, ax, acc2, wh=None, bh=None):
    head = wh is not None

    def body(x_ref, ax_ref, n_ref, w_ref, b_ref, o_ref):
        num = jnp.concatenate([n_ref[0][:, :DH], n_ref[1][:, :DH]], axis=1)
        den = jnp.concatenate([n_ref[0][:, DH:], n_ref[1][:, DH:]], axis=1)
        xn = x_ref[...] + jnp.maximum(ax_ref[...] + num / (den + 1e-6), 0.0)
        if head:
            o_ref[...] = jnp.dot(xn, w_ref[...], preferred_element_type=f32) + b_ref[...]
        else:
            o_ref[...] = xn

    DO = wh.shape[1] if head else D
    if wh is None:
        wh = jnp.zeros((D, D), f32)
        bh = jnp.zeros((D,), f32)
    return pl.pallas_call(
        body, grid=(N // _BN,),
        in_specs=[pl.BlockSpec((_BN, D), lambda i: (i, 0)),
                  pl.BlockSpec((_BN, D), lambda i: (i, 0)),
                  pl.BlockSpec((2, _BN, D), lambda i: (0, i, 0)),
                  pl.BlockSpec((D, DO), lambda i: (0, 0)),
                  pl.BlockSpec((1, DO), lambda i: (0, 0))],
        out_specs=pl.BlockSpec((_BN, DO), lambda i: (i, 0)),
        out_shape=jax.ShapeDtypeStruct((N, DO), f32),
    )(x, ax, acc2, wh, bh.reshape(1, -1))


_EPW = E // NW
_GCH = 80
_GNCH = _EPW // _GCH

_EPT = E // NSUB
_SCH = 80
_SNCH = _EPT // _SCH
_NPAD = 10240
_RPT = _NPAD // NSUB
_ZR = 64


def _sc_gather(dbx, ext, src, dst):
    @functools.partial(
        pl.kernel,
        out_type=(jax.ShapeDtypeStruct((E, 2 * D), f32),
                  jax.ShapeDtypeStruct((E, D), f32)),
        mesh=_sc_mesh(),
        scratch_types=[
            pltpu.VMEM((_GCH,), jnp.int32),
            pltpu.VMEM((_GCH,), jnp.int32),
            pltpu.VMEM((_GCH, 2 * D), f32),
            pltpu.VMEM((_GCH, D), f32),
            pltpu.SemaphoreType.DMA,
            pltpu.SemaphoreType.DMA,
        ],
    )
    def k(dbx_hbm, ext_hbm, src_hbm, dst_hbm, dbs_hbm, exd_hbm,
          sbuf, tbuf, dbuf, ebuf, sem1, sem2):
        wid = lax.axis_index("s") * NCORE + lax.axis_index("c")
        base0 = wid * _EPW

        @pl.loop(0, _GNCH)
        def _(ci):
            base = base0 + ci * _GCH
            pltpu.sync_copy(src_hbm.at[pl.ds(base, _GCH)], sbuf)
            pltpu.sync_copy(dst_hbm.at[pl.ds(base, _GCH)], tbuf)
            cp1 = pltpu.async_copy(dbx_hbm.at[sbuf], dbuf, sem1)
            cp2 = pltpu.async_copy(ext_hbm.at[tbuf], ebuf, sem2)
            cp1.wait()
            cp2.wait()
            pltpu.sync_copy(dbuf, dbs_hbm.at[pl.ds(base, _GCH)])
            pltpu.sync_copy(ebuf, exd_hbm.at[pl.ds(base, _GCH)])

    return k(dbx, ext, src, dst)


def _sc_scatter(comb, dst):
    comb = comb.reshape(2 * E, D)

    @functools.partial(
        pl.kernel,
        out_type=jax.ShapeDtypeStruct((2, _NPAD, D), f32),
        mesh=_sc_mesh(),
        scratch_types=[
            pltpu.VMEM((_SCH,), jnp.int32),
            pltpu.VMEM((_SCH, D), f32),
            pltpu.VMEM((_ZR, D), f32),
            pltpu.VMEM_SHARED((_NPAD, D), f32),
        ],
    )
    def k(comb_hbm, dst_hbm, acc_hbm, ibuf, mbuf, zbuf, acc_sh):
        cid = lax.axis_index("c")
        sid = lax.axis_index("s")
        row0 = sid * _RPT

        @pl.loop(0, _ZR)
        def _(r):
            for j in range(D // 16):
                zbuf[r, pl.ds(j * 16, 16)] = jnp.zeros((16,), f32)

        @pl.loop(0, _RPT // _ZR)
        def _(t):
            pltpu.sync_copy(zbuf, acc_sh.at[pl.ds(row0 + t * _ZR, _ZR)])

        plsc.subcore_barrier()

        base0 = sid * _EPT
        fbase0 = cid * E + base0

        @pl.loop(0, _SNCH)
        def _(ci):
            base = base0 + ci * _SCH
            pltpu.sync_copy(dst_hbm.at[pl.ds(base, _SCH)], ibuf)
            pltpu.sync_copy(comb_hbm.at[pl.ds(fbase0 + ci * _SCH, _SCH)], mbuf)
            pltpu.sync_copy(mbuf, acc_sh.at[ibuf], add=True)

        plsc.subcore_barrier()

        @pl.loop(0, _RPT // _ZR)
        def _(t):
            r = row0 + t * _ZR
            pltpu.sync_copy(acc_sh.at[pl.ds(r, _ZR)], zbuf)
            pltpu.sync_copy(zbuf, acc_hbm.at[cid, pl.ds(r, _ZR)])

    return k(comb, dst)


def kernel(x, edge_index, edge_attr, We, be, Ws, bs, Wh, bh):
    src = edge_index[0]
    dst = edge_index[1]
    ea = jnp.pad(edge_attr, ((0, 0), (0, 1)))
    wep = jnp.pad(We, ((0, 1), (0, 0)))

    e = None
    out = None
    for l in range(3):
        W = Ws[l]
        b = bs[l]
        wn = jnp.concatenate([W[0], W[3], W[1], W[4]], axis=1)
        bn = jnp.concatenate([b[0], b[3], b[1], b[4]])
        ax, dbx, ext = _node_proj(x, wn, bn)
        if l == 0:
            ce = _edge_mm(ea, wep @ W[2], be @ W[2] + b[2])
        else:
            ce = _edge_mm(e, W[2], b[2])
        dbs, exd = _sc_gather(dbx, ext, src, dst)
        comb, e = _edge_elem(ce, dbs, exd, e, ea, wep, be)
        acc2 = _sc_scatter(comb, dst)
        if l < 2:
            x = _x_update(x, ax, acc2)
        else:
            out = _x_update(x, ax, acc2, Wh, bh)
    return out


# speedup vs baseline: 3.3396x; 1.3321x over previous
import functools

import jax
import jax.numpy as jnp
from jax import lax
from jax.experimental import pallas as pl
from jax.experimental.pallas import tpu as pltpu
from jax.experimental.pallas import tpu_sc as plsc

N = 10000
E = 320000
D = 128
DH = D // 2
NCORE = 2
NSUB = 16
NW = NCORE * NSUB

f32 = jnp.float32


def _sc_mesh():
    return plsc.VectorSubcoreMesh(
        core_axis_name="c", subcore_axis_name="s",
        num_cores=NCORE, num_subcores=NSUB)


_BN = 1000
_BE = 2000


def _node_proj(x, wn, bn):
    def body(x_ref, w_ref, b_ref, ax_ref, dbx_ref, ex_ref):
        r = jnp.dot(x_ref[...], w_ref[...], preferred_element_type=f32) + b_ref[...]
        ax_ref[...] = r[:, :D]
        dbx_ref[...] = r[:, D:3 * D]
        ex_ref[...] = r[:, 3 * D:]

    return pl.pallas_call(
        body, grid=(N // _BN,),
        in_specs=[pl.BlockSpec((_BN, D), lambda i: (i, 0)),
                  pl.BlockSpec((D, 4 * D), lambda i: (0, 0)),
                  pl.BlockSpec((1, 4 * D), lambda i: (0, 0))],
        out_specs=[pl.BlockSpec((_BN, D), lambda i: (i, 0)),
                   pl.BlockSpec((_BN, 2 * D), lambda i: (i, 0)),
                   pl.BlockSpec((_BN, D), lambda i: (i, 0))],
        out_shape=[jax.ShapeDtypeStruct((N, D), f32),
                   jax.ShapeDtypeStruct((N, 2 * D), f32),
                   jax.ShapeDtypeStruct((N, D), f32)],
    )(x, wn, bn.reshape(1, -1))


def _edge_mm(e, w, b):
    K = e.shape[1]

    def body(e_ref, w_ref, b_ref, o_ref):
        o_ref[...] = jnp.dot(e_ref[...], w_ref[...], preferred_element_type=f32) + b_ref[...]

    return pl.pallas_call(
        body, grid=(E // _BE,),
        in_specs=[pl.BlockSpec((_BE, K), lambda i: (i, 0)),
                  pl.BlockSpec((K, D), lambda i: (0, 0)),
                  pl.BlockSpec((1, D), lambda i: (0, 0))],
        out_specs=pl.BlockSpec((_BE, D), lambda i: (i, 0)),
        out_shape=jax.ShapeDtypeStruct((E, D), f32),
    )(e, w, b.reshape(1, -1))


def _edge_elem(ce, dbs, exd, eprev, ea, wep, bep):
    layer0 = eprev is None

    def body(ce_ref, dbs_ref, exd_ref, ep_ref, w_ref, b_ref, msg_ref, eo_ref):
        dxs = dbs_ref[:, :D]
        bxs = dbs_ref[:, D:]
        e_new = dxs + exd_ref[...] + ce_ref[...]
        sg = jax.nn.sigmoid(e_new)
        msg = sg * bxs
        if layer0:
            ep = jnp.dot(ep_ref[...], w_ref[...], preferred_element_type=f32) + b_ref[...]
        else:
            ep = ep_ref[...]
        eo_ref[...] = ep + jnp.maximum(e_new, 0.0)
        msg_ref[...] = jnp.stack(
            [jnp.concatenate([msg[:, :DH], sg[:, :DH]], axis=1),
             jnp.concatenate([msg[:, DH:], sg[:, DH:]], axis=1)])

    ep_in = ea if layer0 else eprev
    Kp = ep_in.shape[1]
    return pl.pallas_call(
        body, grid=(E // _BE,),
        in_specs=[pl.BlockSpec((_BE, D), lambda i: (i, 0)),
                  pl.BlockSpec((_BE, 2 * D), lambda i: (i, 0)),
                  pl.BlockSpec((_BE, D), lambda i: (i, 0)),
                  pl.BlockSpec((_BE, Kp), lambda i: (i, 0)),
                  pl.BlockSpec((wep.shape[0], D), lambda i: (0, 0)),
                  pl.BlockSpec((1, D), lambda i: (0, 0))],
        out_specs=[pl.BlockSpec((2, _BE, D), lambda i: (0, i, 0)),
                   pl.BlockSpec((_BE, D), lambda i: (i, 0))],
        out_shape=[jax.ShapeDtypeStruct((2, E, D), f32),
                   jax.ShapeDtypeStruct((E, D), f32)],
    )(ce, dbs, exd, ep_in, wep, bep.reshape(1, -1))


def _x_update(x, ax, acc2, wh=None, bh=None):
    head = wh is not None

    def body(x_ref, ax_ref, n_ref, w_ref, b_ref, o_ref):
        num = jnp.concatenate([n_ref[0][:, :DH], n_ref[1][:, :DH]], axis=1)
        den = jnp.concatenate([n_ref[0][:, DH:], n_ref[1][:, DH:]], axis=1)
        xn = x_ref[...] + jnp.maximum(ax_ref[...] + num / (den + 1e-6), 0.0)
        if head:
            o_ref[...] = jnp.dot(xn, w_ref[...], preferred_element_type=f32) + b_ref[...]
        else:
            o_ref[...] = xn

    DO = wh.shape[1] if head else D
    if wh is None:
        wh = jnp.zeros((D, D), f32)
        bh = jnp.zeros((D,), f32)
    return pl.pallas_call(
        body, grid=(N // _BN,),
        in_specs=[pl.BlockSpec((_BN, D), lambda i: (i, 0)),
                  pl.BlockSpec((_BN, D), lambda i: (i, 0)),
                  pl.BlockSpec((2, _BN, D), lambda i: (0, i, 0)),
                  pl.BlockSpec((D, DO), lambda i: (0, 0)),
                  pl.BlockSpec((1, DO), lambda i: (0, 0))],
        out_specs=pl.BlockSpec((_BN, DO), lambda i: (i, 0)),
        out_shape=jax.ShapeDtypeStruct((N, DO), f32),
    )(x, ax, acc2, wh, bh.reshape(1, -1))


_EPW = E // NW
_GCH = 80
_GNCH = _EPW // _GCH

_EPT = E // NSUB
_SCH = 80
_SNCH = _EPT // _SCH
_SK = 2
_SROUNDS = _SNCH // _SK
_NPAD = 10240
_RPT = _NPAD // NSUB
_ZR = 64


_GK = 4
_GROUNDS = (_GNCH - 1) // _GK


def _sc_gather(dbx, ext, src, dst):
    @functools.partial(
        pl.kernel,
        out_type=(jax.ShapeDtypeStruct((E, 2 * D), f32),
                  jax.ShapeDtypeStruct((E, D), f32)),
        mesh=_sc_mesh(),
        scratch_types=[
            pltpu.VMEM((_GK, _GCH), jnp.int32),
            pltpu.VMEM((_GK, _GCH), jnp.int32),
            pltpu.VMEM((_GK, _GCH, 2 * D), f32),
            pltpu.VMEM((_GK, _GCH, D), f32),
            pltpu.SemaphoreType.DMA,
            pltpu.SemaphoreType.DMA,
            pltpu.SemaphoreType.DMA,
        ],
    )
    def k(dbx_hbm, ext_hbm, src_hbm, dst_hbm, dbs_hbm, exd_hbm,
          sbuf, tbuf, dbuf, ebuf, sem_i, sem_g, sem_w):
        wid = lax.axis_index("s") * NCORE + lax.axis_index("c")
        base0 = wid * _EPW

        def fire_idx(r):
            for j in range(_GK):
                b = base0 + (r * _GK + j) * _GCH
                pltpu.async_copy(src_hbm.at[pl.ds(b, _GCH)], sbuf.at[j], sem_i)
                pltpu.async_copy(dst_hbm.at[pl.ds(b, _GCH)], tbuf.at[j], sem_i)

        def drain_idx():
            for j in range(_GK):
                pltpu.make_async_copy(src_hbm.at[pl.ds(base0, _GCH)], sbuf.at[j], sem_i).wait()
                pltpu.make_async_copy(dst_hbm.at[pl.ds(base0, _GCH)], tbuf.at[j], sem_i).wait()

        def fire_gathers():
            for j in range(_GK):
                pltpu.async_copy(dbx_hbm.at[sbuf.at[j]], dbuf.at[j], sem_g)
                pltpu.async_copy(ext_hbm.at[tbuf.at[j]], ebuf.at[j], sem_g)

        def drain_gathers():
            for j in range(_GK):
                pltpu.make_async_copy(dbx_hbm.at[sbuf.at[j]], dbuf.at[j], sem_g).wait()
                pltpu.make_async_copy(ext_hbm.at[tbuf.at[j]], ebuf.at[j], sem_g).wait()

        def fire_writes(r):
            for j in range(_GK):
                b = base0 + (r * _GK + j) * _GCH
                pltpu.async_copy(dbuf.at[j], dbs_hbm.at[pl.ds(b, _GCH)], sem_w)
                pltpu.async_copy(ebuf.at[j], exd_hbm.at[pl.ds(b, _GCH)], sem_w)

        def drain_writes():
            for j in range(_GK):
                pltpu.make_async_copy(dbuf.at[j], dbs_hbm.at[pl.ds(base0, _GCH)], sem_w).wait()
                pltpu.make_async_copy(ebuf.at[j], exd_hbm.at[pl.ds(base0, _GCH)], sem_w).wait()

        @pl.loop(0, _GROUNDS)
        def _(r):
            fire_idx(r)

            @pl.when(r > 0)
            def _():
                drain_writes()

            drain_idx()
            fire_gathers()
            drain_gathers()
            fire_writes(r)

        drain_writes()
        tb = base0 + _GROUNDS * _GK * _GCH
        pltpu.sync_copy(src_hbm.at[pl.ds(tb, _GCH)], sbuf.at[0])
        pltpu.sync_copy(dst_hbm.at[pl.ds(tb, _GCH)], tbuf.at[0])
        cp1 = pltpu.async_copy(dbx_hbm.at[sbuf.at[0]], dbuf.at[0], sem_g)
        cp2 = pltpu.async_copy(ext_hbm.at[tbuf.at[0]], ebuf.at[0], sem_g)
        cp1.wait()
        cp2.wait()
        pltpu.sync_copy(dbuf.at[0], dbs_hbm.at[pl.ds(tb, _GCH)])
        pltpu.sync_copy(ebuf.at[0], exd_hbm.at[pl.ds(tb, _GCH)])

    return k(dbx, ext, src, dst)


def _sc_scatter(comb, dst):
    comb = comb.reshape(2 * E, D)

    @functools.partial(
        pl.kernel,
        out_type=jax.ShapeDtypeStruct((2, _NPAD, D), f32),
        mesh=_sc_mesh(),
        scratch_types=[
            pltpu.VMEM((2, _SK, _SCH), jnp.int32),
            pltpu.VMEM((2, _SK, _SCH, D), f32),
            pltpu.VMEM((_ZR // 2, D), f32),
            pltpu.VMEM_SHARED((_NPAD, D), f32),
            pltpu.SemaphoreType.DMA,
            pltpu.SemaphoreType.DMA,
        ],
    )
    def k(comb_hbm, dst_hbm, acc_hbm, ibuf, mbuf, zbuf, acc_sh, sem_l0, sem_l1):
        cid = lax.axis_index("c")
        sid = lax.axis_index("s")
        row0 = sid * _RPT

        @pl.loop(0, _ZR // 2)
        def _(r):
            for j in range(D // 16):
                zbuf[r, pl.ds(j * 16, 16)] = jnp.zeros((16,), f32)

        @pl.loop(0, _RPT // (_ZR // 2))
        def _(t):
            pltpu.sync_copy(zbuf, acc_sh.at[pl.ds(row0 + t * (_ZR // 2), _ZR // 2)])

        plsc.subcore_barrier()

        base0 = sid * _EPT
        fbase0 = cid * E + base0
        sems = (sem_l0, sem_l1)

        def fire_group(r, sl):
            for j in range(_SK):
                off = (r * _SK + j) * _SCH
                pltpu.async_copy(dst_hbm.at[pl.ds(base0 + off, _SCH)],
                                 ibuf.at[sl, j], sems[sl])
                pltpu.async_copy(comb_hbm.at[pl.ds(fbase0 + off, _SCH)],
                                 mbuf.at[sl, j], sems[sl])

        def drain_group(sl):
            for j in range(_SK):
                pltpu.make_async_copy(dst_hbm.at[pl.ds(base0, _SCH)],
                                      ibuf.at[sl, j], sems[sl]).wait()
                pltpu.make_async_copy(comb_hbm.at[pl.ds(fbase0, _SCH)],
                                      mbuf.at[sl, j], sems[sl]).wait()

        fire_group(0, 0)

        @pl.loop(0, _SROUNDS - 1, step=2)
        def _(i):
            for sl in range(2):
                r = i + sl
                drain_group(sl)
                fire_group(r + 1, 1 - sl)
                for j in range(_SK):
                    pltpu.sync_copy(mbuf.at[sl, j], acc_sh.at[ibuf.at[sl, j]], add=True)

        drain_group(0)
        for j in range(_SK):
            pltpu.sync_copy(mbuf.at[0, j], acc_sh.at[ibuf.at[0, j]], add=True)

        plsc.subcore_barrier()

        @pl.loop(0, _RPT // (_ZR // 2))
        def _(t):
            r = row0 + t * (_ZR // 2)
            pltpu.sync_copy(acc_sh.at[pl.ds(r, _ZR // 2)], zbuf)
            pltpu.sync_copy(zbuf, acc_hbm.at[cid, pl.ds(r, _ZR // 2)])

    return k(comb, dst)


def kernel(x, edge_index, edge_attr, We, be, Ws, bs, Wh, bh):
    src = edge_index[0]
    dst = edge_index[1]
    ea = jnp.pad(edge_attr, ((0, 0), (0, 1)))
    wep = jnp.pad(We, ((0, 1), (0, 0)))

    e = None
    out = None
    for l in range(3):
        W = Ws[l]
        b = bs[l]
        wn = jnp.concatenate([W[0], W[3], W[1], W[4]], axis=1)
        bn = jnp.concatenate([b[0], b[3], b[1], b[4]])
        ax, dbx, ext = _node_proj(x, wn, bn)
        if l == 0:
            ce = _edge_mm(ea, wep @ W[2], be @ W[2] + b[2])
        else:
            ce = _edge_mm(e, W[2], b[2])
        dbs, exd = _sc_gather(dbx, ext, src, dst)
        comb, e = _edge_elem(ce, dbs, exd, e, ea, wep, be)
        acc2 = _sc_scatter(comb, dst)
        if l < 2:
            x = _x_update(x, ax, acc2)
        else:
            out = _x_update(x, ax, acc2, Wh, bh)
    return out


# speedup vs baseline: 3.8156x; 1.1425x over previous
import functools

import jax
import jax.numpy as jnp
from jax import lax
from jax.experimental import pallas as pl
from jax.experimental.pallas import tpu as pltpu
from jax.experimental.pallas import tpu_sc as plsc

N = 10000
E = 320000
D = 128
DH = D // 2
NCORE = 2
NSUB = 16
NW = NCORE * NSUB

f32 = jnp.float32


def _sc_mesh():
    return plsc.VectorSubcoreMesh(
        core_axis_name="c", subcore_axis_name="s",
        num_cores=NCORE, num_subcores=NSUB)


_BN = 1000
_BE = 2000


def _node_proj(x, wn, bn):
    def body(x_ref, w_ref, b_ref, ax_ref, dbx_ref, ex_ref):
        r = jnp.dot(x_ref[...], w_ref[...], preferred_element_type=f32) + b_ref[...]
        ax_ref[...] = r[:, :D]
        dbx_ref[...] = r[:, D:3 * D]
        ex_ref[...] = r[:, 3 * D:]

    return pl.pallas_call(
        body, grid=(N // _BN,),
        in_specs=[pl.BlockSpec((_BN, D), lambda i: (i, 0)),
                  pl.BlockSpec((D, 4 * D), lambda i: (0, 0)),
                  pl.BlockSpec((1, 4 * D), lambda i: (0, 0))],
        out_specs=[pl.BlockSpec((_BN, D), lambda i: (i, 0)),
                   pl.BlockSpec((_BN, 2 * D), lambda i: (i, 0)),
                   pl.BlockSpec((_BN, D), lambda i: (i, 0))],
        out_shape=[jax.ShapeDtypeStruct((N, D), f32),
                   jax.ShapeDtypeStruct((N, 2 * D), f32),
                   jax.ShapeDtypeStruct((N, D), f32)],
    )(x, wn, bn.reshape(1, -1))


def _edge_elem(dbs, exd, eprev, ea, wep, bep, w2, b2):
    layer0 = eprev is None

    def body(dbs_ref, exd_ref, ep_ref, w_ref, b_ref, w2_ref, b2_ref, msg_ref, eo_ref):
        dxs = dbs_ref[:, :D]
        bxs = dbs_ref[:, D:]
        if layer0:
            ep = jnp.dot(ep_ref[...], w_ref[...], preferred_element_type=f32) + b_ref[...]
        else:
            ep = ep_ref[...]
        ce = jnp.dot(ep, w2_ref[...], preferred_element_type=f32) + b2_ref[...]
        e_new = dxs + exd_ref[...] + ce
        sg = jax.nn.sigmoid(e_new)
        msg = sg * bxs
        eo_ref[...] = ep + jnp.maximum(e_new, 0.0)
        msg_ref[...] = jnp.stack(
            [jnp.concatenate([msg[:, :DH], sg[:, :DH]], axis=1),
             jnp.concatenate([msg[:, DH:], sg[:, DH:]], axis=1)])

    ep_in = ea if layer0 else eprev
    Kp = ep_in.shape[1]
    return pl.pallas_call(
        body, grid=(E // _BE,),
        in_specs=[pl.BlockSpec((_BE, 2 * D), lambda i: (i, 0)),
                  pl.BlockSpec((_BE, D), lambda i: (i, 0)),
                  pl.BlockSpec((_BE, Kp), lambda i: (i, 0)),
                  pl.BlockSpec((wep.shape[0], D), lambda i: (0, 0)),
                  pl.BlockSpec((1, D), lambda i: (0, 0)),
                  pl.BlockSpec((D, D), lambda i: (0, 0)),
                  pl.BlockSpec((1, D), lambda i: (0, 0))],
        out_specs=[pl.BlockSpec((2, _BE, D), lambda i: (0, i, 0)),
                   pl.BlockSpec((_BE, D), lambda i: (i, 0))],
        out_shape=[jax.ShapeDtypeStruct((2, E, D), f32),
                   jax.ShapeDtypeStruct((E, D), f32)],
    )(dbs, exd, ep_in, wep, bep.reshape(1, -1), w2, b2.reshape(1, -1))


def _x_update(x, ax, acc2, wh=None, bh=None):
    head = wh is not None

    def body(x_ref, ax_ref, n_ref, w_ref, b_ref, o_ref):
        num = jnp.concatenate([n_ref[0][:, :DH], n_ref[1][:, :DH]], axis=1)
        den = jnp.concatenate([n_ref[0][:, DH:], n_ref[1][:, DH:]], axis=1)
        xn = x_ref[...] + jnp.maximum(ax_ref[...] + num / (den + 1e-6), 0.0)
        if head:
            o_ref[...] = jnp.dot(xn, w_ref[...], preferred_element_type=f32) + b_ref[...]
        else:
            o_ref[...] = xn

    DO = wh.shape[1] if head else D
    if wh is None:
        wh = jnp.zeros((D, D), f32)
        bh = jnp.zeros((D,), f32)
    return pl.pallas_call(
        body, grid=(N // _BN,),
        in_specs=[pl.BlockSpec((_BN, D), lambda i: (i, 0)),
                  pl.BlockSpec((_BN, D), lambda i: (i, 0)),
                  pl.BlockSpec((2, _BN, D), lambda i: (0, i, 0)),
                  pl.BlockSpec((D, DO), lambda i: (0, 0)),
                  pl.BlockSpec((1, DO), lambda i: (0, 0))],
        out_specs=pl.BlockSpec((_BN, DO), lambda i: (i, 0)),
        out_shape=jax.ShapeDtypeStruct((N, DO), f32),
    )(x, ax, acc2, wh, bh.reshape(1, -1))


_EPW = E // NW
_GCH = 80
_GNCH = _EPW // _GCH

_EPT = E // NSUB
_SCH = 80
_SNCH = _EPT // _SCH
_SK = 2
_SROUNDS = _SNCH // _SK
_NPAD = 10240
_RPT = _NPAD // NSUB
_ZR = 64


_GK = 4
_GROUNDS = (_GNCH - 1) // _GK


def _sc_gather(dbx, ext, src, dst):
    @functools.partial(
        pl.kernel,
        out_type=(jax.ShapeDtypeStruct((E, 2 * D), f32),
                  jax.ShapeDtypeStruct((E, D), f32)),
        mesh=_sc_mesh(),
        scratch_types=[
            pltpu.VMEM((_GK, _GCH), jnp.int32),
            pltpu.VMEM((_GK, _GCH), jnp.int32),
            pltpu.VMEM((_GK, _GCH, 2 * D), f32),
            pltpu.VMEM((_GK, _GCH, D), f32),
            pltpu.SemaphoreType.DMA,
            pltpu.SemaphoreType.DMA,
            pltpu.SemaphoreType.DMA,
        ],
    )
    def k(dbx_hbm, ext_hbm, src_hbm, dst_hbm, dbs_hbm, exd_hbm,
          sbuf, tbuf, dbuf, ebuf, sem_i, sem_g, sem_w):
        wid = lax.axis_index("s") * NCORE + lax.axis_index("c")
        base0 = wid * _EPW

        def fire_idx(r):
            for j in range(_GK):
                b = base0 + (r * _GK + j) * _GCH
                pltpu.async_copy(src_hbm.at[pl.ds(b, _GCH)], sbuf.at[j], sem_i)
                pltpu.async_copy(dst_hbm.at[pl.ds(b, _GCH)], tbuf.at[j], sem_i)

        def drain_idx():
            for j in range(_GK):
                pltpu.make_async_copy(src_hbm.at[pl.ds(base0, _GCH)], sbuf.at[j], sem_i).wait()
                pltpu.make_async_copy(dst_hbm.at[pl.ds(base0, _GCH)], tbuf.at[j], sem_i).wait()

        def fire_gathers():
            for j in range(_GK):
                pltpu.async_copy(dbx_hbm.at[sbuf.at[j]], dbuf.at[j], sem_g)
                pltpu.async_copy(ext_hbm.at[tbuf.at[j]], ebuf.at[j], sem_g)

        def drain_gathers():
            for j in range(_GK):
                pltpu.make_async_copy(dbx_hbm.at[sbuf.at[j]], dbuf.at[j], sem_g).wait()
                pltpu.make_async_copy(ext_hbm.at[tbuf.at[j]], ebuf.at[j], sem_g).wait()

        def fire_writes(r):
            for j in range(_GK):
                b = base0 + (r * _GK + j) * _GCH
                pltpu.async_copy(dbuf.at[j], dbs_hbm.at[pl.ds(b, _GCH)], sem_w)
                pltpu.async_copy(ebuf.at[j], exd_hbm.at[pl.ds(b, _GCH)], sem_w)

        def drain_writes():
            for j in range(_GK):
                pltpu.make_async_copy(dbuf.at[j], dbs_hbm.at[pl.ds(base0, _GCH)], sem_w).wait()
                pltpu.make_async_copy(ebuf.at[j], exd_hbm.at[pl.ds(base0, _GCH)], sem_w).wait()

        @pl.loop(0, _GROUNDS)
        def _(r):
            fire_idx(r)

            @pl.when(r > 0)
            def _():
                drain_writes()

            drain_idx()
            fire_gathers()
            drain_gathers()
            fire_writes(r)

        drain_writes()
        tb = base0 + _GROUNDS * _GK * _GCH
        pltpu.sync_copy(src_hbm.at[pl.ds(tb, _GCH)], sbuf.at[0])
        pltpu.sync_copy(dst_hbm.at[pl.ds(tb, _GCH)], tbuf.at[0])
        cp1 = pltpu.async_copy(dbx_hbm.at[sbuf.at[0]], dbuf.at[0], sem_g)
        cp2 = pltpu.async_copy(ext_hbm.at[tbuf.at[0]], ebuf.at[0], sem_g)
        cp1.wait()
        cp2.wait()
        pltpu.sync_copy(dbuf.at[0], dbs_hbm.at[pl.ds(tb, _GCH)])
        pltpu.sync_copy(ebuf.at[0], exd_hbm.at[pl.ds(tb, _GCH)])

    return k(dbx, ext, src, dst)


def _sc_scatter(comb, dst):
    comb = comb.reshape(2 * E, D)

    @functools.partial(
        pl.kernel,
        out_type=jax.ShapeDtypeStruct((2, _NPAD, D), f32),
        mesh=_sc_mesh(),
        scratch_types=[
            pltpu.VMEM((2, _SK, _SCH), jnp.int32),
            pltpu.VMEM((2, _SK, _SCH, D), f32),
            pltpu.VMEM((_ZR // 2, D), f32),
            pltpu.VMEM_SHARED((_NPAD, D), f32),
            pltpu.SemaphoreType.DMA,
            pltpu.SemaphoreType.DMA,
        ],
    )
    def k(comb_hbm, dst_hbm, acc_hbm, ibuf, mbuf, zbuf, acc_sh, sem_l0, sem_l1):
        cid = lax.axis_index("c")
        sid = lax.axis_index("s")
        row0 = sid * _RPT

        @pl.loop(0, _ZR // 2)
        def _(r):
            for j in range(D // 16):
                zbuf[r, pl.ds(j * 16, 16)] = jnp.zeros((16,), f32)

        @pl.loop(0, _RPT // (_ZR // 2))
        def _(t):
            pltpu.sync_copy(zbuf, acc_sh.at[pl.ds(row0 + t * (_ZR // 2), _ZR // 2)])

        plsc.subcore_barrier()

        base0 = sid * _EPT
        fbase0 = cid * E + base0
        sems = (sem_l0, sem_l1)

        def fire_group(r, sl):
            for j in range(_SK):
                off = (r * _SK + j) * _SCH
                pltpu.async_copy(dst_hbm.at[pl.ds(base0 + off, _SCH)],
                                 ibuf.at[sl, j], sems[sl])
                pltpu.async_copy(comb_hbm.at[pl.ds(fbase0 + off, _SCH)],
                                 mbuf.at[sl, j], sems[sl])

        def drain_group(sl):
            for j in range(_SK):
                pltpu.make_async_copy(dst_hbm.at[pl.ds(base0, _SCH)],
                                      ibuf.at[sl, j], sems[sl]).wait()
                pltpu.make_async_copy(comb_hbm.at[pl.ds(fbase0, _SCH)],
                                      mbuf.at[sl, j], sems[sl]).wait()

        fire_group(0, 0)

        @pl.loop(0, _SROUNDS - 1, step=2)
        def _(i):
            for sl in range(2):
                r = i + sl
                drain_group(sl)
                fire_group(r + 1, 1 - sl)
                for j in range(_SK):
                    pltpu.sync_copy(mbuf.at[sl, j], acc_sh.at[ibuf.at[sl, j]], add=True)

        drain_group(0)
        for j in range(_SK):
            pltpu.sync_copy(mbuf.at[0, j], acc_sh.at[ibuf.at[0, j]], add=True)

        plsc.subcore_barrier()

        @pl.loop(0, _RPT // (_ZR // 2))
        def _(t):
            r = row0 + t * (_ZR // 2)
            pltpu.sync_copy(acc_sh.at[pl.ds(r, _ZR // 2)], zbuf)
            pltpu.sync_copy(zbuf, acc_hbm.at[cid, pl.ds(r, _ZR // 2)])

    return k(comb, dst)


def kernel(x, edge_index, edge_attr, We, be, Ws, bs, Wh, bh):
    src = edge_index[0]
    dst = edge_index[1]
    ea = jnp.pad(edge_attr, ((0, 0), (0, 1)))
    wep = jnp.pad(We, ((0, 1), (0, 0)))

    e = None
    out = None
    for l in range(3):
        W = Ws[l]
        b = bs[l]
        wn = jnp.concatenate([W[0], W[3], W[1], W[4]], axis=1)
        bn = jnp.concatenate([b[0], b[3], b[1], b[4]])
        ax, dbx, ext = _node_proj(x, wn, bn)
        dbs, exd = _sc_gather(dbx, ext, src, dst)
        comb, e = _edge_elem(dbs, exd, e, ea, wep, be, W[2], b[2])
        acc2 = _sc_scatter(comb, dst)
        if l < 2:
            x = _x_update(x, ax, acc2)
        else:
            out = _x_update(x, ax, acc2, Wh, bh)
    return out


# speedup vs baseline: 4.3573x; 1.1420x over previous
import functools

import jax
import jax.numpy as jnp
from jax import lax
from jax.experimental import pallas as pl
from jax.experimental.pallas import tpu as pltpu
from jax.experimental.pallas import tpu_sc as plsc

N = 10000
E = 320000
D = 128
DH = D // 2
NCORE = 2
NSUB = 16
NW = NCORE * NSUB

f32 = jnp.float32
bf16 = jnp.bfloat16


def _sc_mesh():
    return plsc.VectorSubcoreMesh(
        core_axis_name="c", subcore_axis_name="s",
        num_cores=NCORE, num_subcores=NSUB)


_BN = 1000
_BE = 2000


def _bf16_bits(x):
    u = lax.bitcast_convert_type(x, jnp.uint32)
    r = u + jnp.uint32(0x7FFF) + ((u >> 16) & jnp.uint32(1))
    return r >> 16


def _pack2(a, b):
    return lax.bitcast_convert_type(_bf16_bits(a) | (_bf16_bits(b) << 16), jnp.int32)


def _unpack_lo(w):
    return lax.bitcast_convert_type(lax.shift_left(w, 16), f32)


def _unpack_hi(w):
    return lax.bitcast_convert_type(w & jnp.int32(-65536), f32)


def _node_proj(x, wn, bn):
    def body(x_ref, w_ref, b_ref, ax_ref, dbx_ref, ex_ref):
        r = jnp.dot(x_ref[...], w_ref[...], preferred_element_type=f32) + b_ref[...]
        ax_ref[...] = r[:, :D]
        dx = r[:, D:2 * D]
        bx = r[:, 2 * D:3 * D]
        ex = r[:, 3 * D:]
        dbx_ref[...] = jnp.concatenate(
            [_pack2(dx[:, :DH], dx[:, DH:]), _pack2(bx[:, :DH], bx[:, DH:])], axis=1)
        ex_ref[...] = ex

    return pl.pallas_call(
        body, grid=(N // _BN,),
        in_specs=[pl.BlockSpec((_BN, D), lambda i: (i, 0)),
                  pl.BlockSpec((D, 4 * D), lambda i: (0, 0)),
                  pl.BlockSpec((1, 4 * D), lambda i: (0, 0))],
        out_specs=[pl.BlockSpec((_BN, D), lambda i: (i, 0)),
                   pl.BlockSpec((_BN, D), lambda i: (i, 0)),
                   pl.BlockSpec((_BN, D), lambda i: (i, 0))],
        out_shape=[jax.ShapeDtypeStruct((N, D), f32),
                   jax.ShapeDtypeStruct((N, D), jnp.int32),
                   jax.ShapeDtypeStruct((N, D), f32)],
    )(x, wn, bn.reshape(1, -1))


def _edge_elem(dbs, exd, eprev, ea, wep, bep, w2, b2):
    layer0 = eprev is None

    def body(dbs_ref, exd_ref, ep_ref, w_ref, b_ref, w2_ref, b2_ref, msg_ref, eo_ref):
        dxw = dbs_ref[:, :DH]
        bxw = dbs_ref[:, DH:]
        dxs = jnp.concatenate([_unpack_lo(dxw), _unpack_hi(dxw)], axis=1)
        bxs = jnp.concatenate([_unpack_lo(bxw), _unpack_hi(bxw)], axis=1)
        exd = exd_ref[...]
        if layer0:
            ep = jnp.dot(ep_ref[...], w_ref[...], preferred_element_type=f32) + b_ref[...]
        else:
            ep = ep_ref[...]
        ce = jnp.dot(ep, w2_ref[...], preferred_element_type=f32) + b2_ref[...]
        e_new = dxs + exd + ce
        sg = jax.nn.sigmoid(e_new)
        msg = sg * bxs
        eo_ref[...] = ep + jnp.maximum(e_new, 0.0)
        msg_ref[...] = jnp.stack(
            [jnp.concatenate([msg[:, :DH], sg[:, :DH]], axis=1),
             jnp.concatenate([msg[:, DH:], sg[:, DH:]], axis=1)])

    ep_in = ea if layer0 else eprev
    Kp = ep_in.shape[1]
    return pl.pallas_call(
        body, grid=(E // _BE,),
        in_specs=[pl.BlockSpec((_BE, D), lambda i: (i, 0)),
                  pl.BlockSpec((_BE, D), lambda i: (i, 0)),
                  pl.BlockSpec((_BE, Kp), lambda i: (i, 0)),
                  pl.BlockSpec((wep.shape[0], D), lambda i: (0, 0)),
                  pl.BlockSpec((1, D), lambda i: (0, 0)),
                  pl.BlockSpec((D, D), lambda i: (0, 0)),
                  pl.BlockSpec((1, D), lambda i: (0, 0))],
        out_specs=[pl.BlockSpec((2, _BE, D), lambda i: (0, i, 0)),
                   pl.BlockSpec((_BE, D), lambda i: (i, 0))],
        out_shape=[jax.ShapeDtypeStruct((2, E, D), f32),
                   jax.ShapeDtypeStruct((E, D), f32)],
    )(dbs, exd, ep_in, wep, bep.reshape(1, -1), w2, b2.reshape(1, -1))


def _x_update(x, ax, acc2, wh=None, bh=None):
    head = wh is not None

    def body(x_ref, ax_ref, n_ref, w_ref, b_ref, o_ref):
        num = jnp.concatenate([n_ref[0][:, :DH], n_ref[1][:, :DH]], axis=1)
        den = jnp.concatenate([n_ref[0][:, DH:], n_ref[1][:, DH:]], axis=1)
        xn = x_ref[...] + jnp.maximum(ax_ref[...] + num / (den + 1e-6), 0.0)
        if head:
            o_ref[...] = jnp.dot(xn, w_ref[...], preferred_element_type=f32) + b_ref[...]
        else:
            o_ref[...] = xn

    DO = wh.shape[1] if head else D
    if wh is None:
        wh = jnp.zeros((D, D), f32)
        bh = jnp.zeros((D,), f32)
    return pl.pallas_call(
        body, grid=(N // _BN,),
        in_specs=[pl.BlockSpec((_BN, D), lambda i: (i, 0)),
                  pl.BlockSpec((_BN, D), lambda i: (i, 0)),
                  pl.BlockSpec((2, _BN, D), lambda i: (0, i, 0)),
                  pl.BlockSpec((D, DO), lambda i: (0, 0)),
                  pl.BlockSpec((1, DO), lambda i: (0, 0))],
        out_specs=pl.BlockSpec((_BN, DO), lambda i: (i, 0)),
        out_shape=jax.ShapeDtypeStruct((N, DO), f32),
    )(x, ax, acc2, wh, bh.reshape(1, -1))


_EPW = E // NW
_GCH = 80
_GNCH = _EPW // _GCH

_EPT = E // NSUB
_SCH = 80
_SNCH = _EPT // _SCH
_SK = 2
_SROUNDS = _SNCH // _SK
_NPAD = 10240
_RPT = _NPAD // NSUB
_ZR = 64


_GK = 4
_GROUNDS = (_GNCH - 1) // _GK


def _sc_gather(dbx, ext, src, dst):
    @functools.partial(
        pl.kernel,
        out_type=(jax.ShapeDtypeStruct((E, D), jnp.int32),
                  jax.ShapeDtypeStruct((E, D), f32)),
        mesh=_sc_mesh(),
        scratch_types=[
            pltpu.VMEM((_GK, _GCH), jnp.int32),
            pltpu.VMEM((_GK, _GCH), jnp.int32),
            pltpu.VMEM((_GK, _GCH, D), jnp.int32),
            pltpu.VMEM((_GK, _GCH, D), f32),
            pltpu.SemaphoreType.DMA,
            pltpu.SemaphoreType.DMA,
            pltpu.SemaphoreType.DMA,
        ],
    )
    def k(dbx_hbm, ext_hbm, src_hbm, dst_hbm, dbs_hbm, exd_hbm,
          sbuf, tbuf, dbuf, ebuf, sem_i, sem_g, sem_w):
        wid = lax.axis_index("s") * NCORE + lax.axis_index("c")
        base0 = wid * _EPW

        def fire_idx(r):
            for j in range(_GK):
                b = base0 + (r * _GK + j) * _GCH
                pltpu.async_copy(src_hbm.at[pl.ds(b, _GCH)], sbuf.at[j], sem_i)
                pltpu.async_copy(dst_hbm.at[pl.ds(b, _GCH)], tbuf.at[j], sem_i)

        def drain_idx():
            for j in range(_GK):
                pltpu.make_async_copy(src_hbm.at[pl.ds(base0, _GCH)], sbuf.at[j], sem_i).wait()
                pltpu.make_async_copy(dst_hbm.at[pl.ds(base0, _GCH)], tbuf.at[j], sem_i).wait()

        def fire_gathers():
            for j in range(_GK):
                pltpu.async_copy(dbx_hbm.at[sbuf.at[j]], dbuf.at[j], sem_g)
                pltpu.async_copy(ext_hbm.at[tbuf.at[j]], ebuf.at[j], sem_g)

        def drain_gathers():
            for j in range(_GK):
                pltpu.make_async_copy(dbx_hbm.at[sbuf.at[j]], dbuf.at[j], sem_g).wait()
                pltpu.make_async_copy(ext_hbm.at[tbuf.at[j]], ebuf.at[j], sem_g).wait()

        def fire_writes(r):
            for j in range(_GK):
                b = base0 + (r * _GK + j) * _GCH
                pltpu.async_copy(dbuf.at[j], dbs_hbm.at[pl.ds(b, _GCH)], sem_w)
                pltpu.async_copy(ebuf.at[j], exd_hbm.at[pl.ds(b, _GCH)], sem_w)

        def drain_writes():
            for j in range(_GK):
                pltpu.make_async_copy(dbuf.at[j], dbs_hbm.at[pl.ds(base0, _GCH)], sem_w).wait()
                pltpu.make_async_copy(ebuf.at[j], exd_hbm.at[pl.ds(base0, _GCH)], sem_w).wait()

        @pl.loop(0, _GROUNDS)
        def _(r):
            fire_idx(r)

            @pl.when(r > 0)
            def _():
                drain_writes()

            drain_idx()
            fire_gathers()
            drain_gathers()
            fire_writes(r)

        drain_writes()
        tb = base0 + _GROUNDS * _GK * _GCH
        pltpu.sync_copy(src_hbm.at[pl.ds(tb, _GCH)], sbuf.at[0])
        pltpu.sync_copy(dst_hbm.at[pl.ds(tb, _GCH)], tbuf.at[0])
        cp1 = pltpu.async_copy(dbx_hbm.at[sbuf.at[0]], dbuf.at[0], sem_g)
        cp2 = pltpu.async_copy(ext_hbm.at[tbuf.at[0]], ebuf.at[0], sem_g)
        cp1.wait()
        cp2.wait()
        pltpu.sync_copy(dbuf.at[0], dbs_hbm.at[pl.ds(tb, _GCH)])
        pltpu.sync_copy(ebuf.at[0], exd_hbm.at[pl.ds(tb, _GCH)])

    return k(dbx, ext, src, dst)


def _sc_scatter(comb, dst):
    comb = comb.reshape(2 * E, D)

    @functools.partial(
        pl.kernel,
        out_type=jax.ShapeDtypeStruct((2, _NPAD, D), f32),
        mesh=_sc_mesh(),
        scratch_types=[
            pltpu.VMEM((2, _SK, _SCH), jnp.int32),
            pltpu.VMEM((2, _SK, _SCH, D), f32),
            pltpu.VMEM((_ZR // 2, D), f32),
            pltpu.VMEM_SHARED((_NPAD, D), f32),
            pltpu.SemaphoreType.DMA,
            pltpu.SemaphoreType.DMA,
        ],
    )
    def k(comb_hbm, dst_hbm, acc_hbm, ibuf, mbuf, zbuf, acc_sh, sem_l0, sem_l1):
        cid = lax.axis_index("c")
        sid = lax.axis_index("s")
        row0 = sid * _RPT

        @pl.loop(0, _ZR // 2)
        def _(r):
            for j in range(D // 16):
                zbuf[r, pl.ds(j * 16, 16)] = jnp.zeros((16,), f32)

        @pl.loop(0, _RPT // (_ZR // 2))
        def _(t):
            pltpu.sync_copy(zbuf, acc_sh.at[pl.ds(row0 + t * (_ZR // 2), _ZR // 2)])

        plsc.subcore_barrier()

        base0 = sid * _EPT
        fbase0 = cid * E + base0
        sems = (sem_l0, sem_l1)

        def fire_group(r, sl):
            for j in range(_SK):
                off = (r * _SK + j) * _SCH
                pltpu.async_copy(dst_hbm.at[pl.ds(base0 + off, _SCH)],
                                 ibuf.at[sl, j], sems[sl])
                pltpu.async_copy(comb_hbm.at[pl.ds(fbase0 + off, _SCH)],
                                 mbuf.at[sl, j], sems[sl])

        def drain_group(sl):
            for j in range(_SK):
                pltpu.make_async_copy(dst_hbm.at[pl.ds(base0, _SCH)],
                                      ibuf.at[sl, j], sems[sl]).wait()
                pltpu.make_async_copy(comb_hbm.at[pl.ds(fbase0, _SCH)],
                                      mbuf.at[sl, j], sems[sl]).wait()

        fire_group(0, 0)

        @pl.loop(0, _SROUNDS - 1, step=2)
        def _(i):
            for sl in range(2):
                r = i + sl
                drain_group(sl)
                fire_group(r + 1, 1 - sl)
                for j in range(_SK):
                    pltpu.sync_copy(mbuf.at[sl, j], acc_sh.at[ibuf.at[sl, j]], add=True)

        drain_group(0)
        for j in range(_SK):
            pltpu.sync_copy(mbuf.at[0, j], acc_sh.at[ibuf.at[0, j]], add=True)

        plsc.subcore_barrier()

        @pl.loop(0, _RPT // (_ZR // 2))
        def _(t):
            r = row0 + t * (_ZR // 2)
            pltpu.sync_copy(acc_sh.at[pl.ds(r, _ZR // 2)], zbuf)
            pltpu.sync_copy(zbuf, acc_hbm.at[cid, pl.ds(r, _ZR // 2)])

    return k(comb, dst)


def kernel(x, edge_index, edge_attr, We, be, Ws, bs, Wh, bh):
    src = edge_index[0]
    dst = edge_index[1]
    ea = jnp.pad(edge_attr, ((0, 0), (0, 1)))
    wep = jnp.pad(We, ((0, 1), (0, 0)))

    e = None
    out = None
    for l in range(3):
        W = Ws[l]
        b = bs[l]
        wn = jnp.concatenate([W[0], W[3], W[1], W[4]], axis=1)
        bn = jnp.concatenate([b[0], b[3], b[1], b[4]])
        ax, dbx, ext = _node_proj(x, wn, bn)
        dbs, exd = _sc_gather(dbx, ext, src, dst)
        comb, e = _edge_elem(dbs, exd, e, ea, wep, be, W[2], b[2])
        acc2 = _sc_scatter(comb, dst)
        if l < 2:
            x = _x_update(x, ax, acc2)
        else:
            out = _x_update(x, ax, acc2, Wh, bh)
    return out


# speedup vs baseline: 4.3794x; 1.0051x over previous
import functools

import jax
import jax.numpy as jnp
from jax import lax
from jax.experimental import pallas as pl
from jax.experimental.pallas import tpu as pltpu
from jax.experimental.pallas import tpu_sc as plsc

N = 10000
E = 320000
D = 128
DH = D // 2
NCORE = 2
NSUB = 16
NW = NCORE * NSUB

f32 = jnp.float32
bf16 = jnp.bfloat16


def _sc_mesh():
    return plsc.VectorSubcoreMesh(
        core_axis_name="c", subcore_axis_name="s",
        num_cores=NCORE, num_subcores=NSUB)


_BN = 1000
_BE = 2000


def _bf16_bits(x):
    u = lax.bitcast_convert_type(x, jnp.uint32)
    r = u + jnp.uint32(0x7FFF) + ((u >> 16) & jnp.uint32(1))
    return r >> 16


def _pack2(a, b):
    return lax.bitcast_convert_type(_bf16_bits(a) | (_bf16_bits(b) << 16), jnp.int32)


def _unpack_lo(w):
    return lax.bitcast_convert_type(lax.shift_left(w, 16), f32)


def _unpack_hi(w):
    return lax.bitcast_convert_type(w & jnp.int32(-65536), f32)


def _node_proj(x, wn, bn):
    def body(x_ref, w_ref, b_ref, ax_ref, dbx_ref, ex_ref):
        r = jnp.dot(x_ref[...], w_ref[...], preferred_element_type=f32) + b_ref[...]
        ax_ref[...] = r[:, :D]
        dx = r[:, D:2 * D]
        bx = r[:, 2 * D:3 * D]
        ex = r[:, 3 * D:]
        dbx_ref[...] = jnp.concatenate(
            [_pack2(dx[:, :DH], dx[:, DH:]), _pack2(bx[:, :DH], bx[:, DH:])], axis=1)
        ex_ref[...] = ex

    return pl.pallas_call(
        body, grid=(N // _BN,),
        in_specs=[pl.BlockSpec((_BN, D), lambda i: (i, 0)),
                  pl.BlockSpec((D, 4 * D), lambda i: (0, 0)),
                  pl.BlockSpec((1, 4 * D), lambda i: (0, 0))],
        out_specs=[pl.BlockSpec((_BN, D), lambda i: (i, 0)),
                   pl.BlockSpec((_BN, D), lambda i: (i, 0)),
                   pl.BlockSpec((_BN, D), lambda i: (i, 0))],
        out_shape=[jax.ShapeDtypeStruct((N, D), f32),
                   jax.ShapeDtypeStruct((N, D), jnp.int32),
                   jax.ShapeDtypeStruct((N, D), f32)],
    )(x, wn, bn.reshape(1, -1))


def _edge_elem(dbs, exd, eprev, ea, wep, bep, w2, b2):
    layer0 = eprev is None

    def body(dbs_ref, exd_ref, ep_ref, w_ref, b_ref, w2_ref, b2_ref, msg_ref, eo_ref):
        dxw = dbs_ref[:, :DH]
        bxw = dbs_ref[:, DH:]
        dxs = jnp.concatenate([_unpack_lo(dxw), _unpack_hi(dxw)], axis=1)
        bxs = jnp.concatenate([_unpack_lo(bxw), _unpack_hi(bxw)], axis=1)
        exd = exd_ref[...]
        if layer0:
            ep = jnp.dot(ep_ref[...], w_ref[...], preferred_element_type=f32) + b_ref[...]
        else:
            ep = ep_ref[...]
        ce = jnp.dot(ep, w2_ref[...], preferred_element_type=f32) + b2_ref[...]
        e_new = dxs + exd + ce
        sg = jax.nn.sigmoid(e_new)
        msg = sg * bxs
        eo_ref[...] = ep + jnp.maximum(e_new, 0.0)
        msg_ref[...] = jnp.stack(
            [jnp.concatenate([msg[:, :DH], sg[:, :DH]], axis=1),
             jnp.concatenate([msg[:, DH:], sg[:, DH:]], axis=1)])

    ep_in = ea if layer0 else eprev
    Kp = ep_in.shape[1]
    return pl.pallas_call(
        body, grid=(E // _BE,),
        in_specs=[pl.BlockSpec((_BE, D), lambda i: (i, 0)),
                  pl.BlockSpec((_BE, D), lambda i: (i, 0)),
                  pl.BlockSpec((_BE, Kp), lambda i: (i, 0)),
                  pl.BlockSpec((wep.shape[0], D), lambda i: (0, 0)),
                  pl.BlockSpec((1, D), lambda i: (0, 0)),
                  pl.BlockSpec((D, D), lambda i: (0, 0)),
                  pl.BlockSpec((1, D), lambda i: (0, 0))],
        out_specs=[pl.BlockSpec((2, _BE, D), lambda i: (0, i, 0)),
                   pl.BlockSpec((_BE, D), lambda i: (i, 0))],
        out_shape=[jax.ShapeDtypeStruct((2, E, D), f32),
                   jax.ShapeDtypeStruct((E, D), f32)],
    )(dbs, exd, ep_in, wep, bep.reshape(1, -1), w2, b2.reshape(1, -1))


def _x_update(x, ax, acc2, wh=None, bh=None):
    head = wh is not None

    def body(x_ref, ax_ref, n_ref, w_ref, b_ref, o_ref):
        num = jnp.concatenate([n_ref[0][:, :DH], n_ref[1][:, :DH]], axis=1)
        den = jnp.concatenate([n_ref[0][:, DH:], n_ref[1][:, DH:]], axis=1)
        xn = x_ref[...] + jnp.maximum(ax_ref[...] + num / (den + 1e-6), 0.0)
        if head:
            o_ref[...] = jnp.dot(xn, w_ref[...], preferred_element_type=f32) + b_ref[...]
        else:
            o_ref[...] = xn

    DO = wh.shape[1] if head else D
    if wh is None:
        wh = jnp.zeros((D, D), f32)
        bh = jnp.zeros((D,), f32)
    return pl.pallas_call(
        body, grid=(N // _BN,),
        in_specs=[pl.BlockSpec((_BN, D), lambda i: (i, 0)),
                  pl.BlockSpec((_BN, D), lambda i: (i, 0)),
                  pl.BlockSpec((2, _BN, D), lambda i: (0, i, 0)),
                  pl.BlockSpec((D, DO), lambda i: (0, 0)),
                  pl.BlockSpec((1, DO), lambda i: (0, 0))],
        out_specs=pl.BlockSpec((_BN, DO), lambda i: (i, 0)),
        out_shape=jax.ShapeDtypeStruct((N, DO), f32),
    )(x, ax, acc2, wh, bh.reshape(1, -1))


_EPW = E // NW
_GCH = 80
_GNCH = _EPW // _GCH

_EPT = E // NSUB
_SCH = 80
_SNCH = _EPT // _SCH
_SK = 2
_SROUNDS = _SNCH // _SK
_NPAD = 10240
_RPT = _NPAD // NSUB
_ZR = 64


_GK = 5
_GROUNDS = _GNCH // _GK


def _sc_gather(dbx, ext, src, dst):
    @functools.partial(
        pl.kernel,
        out_type=(jax.ShapeDtypeStruct((E, D), jnp.int32),
                  jax.ShapeDtypeStruct((E, D), f32)),
        mesh=_sc_mesh(),
        scratch_types=[
            pltpu.VMEM((_GK, _GCH), jnp.int32),
            pltpu.VMEM((_GK, _GCH), jnp.int32),
            pltpu.VMEM((_GK, _GCH, D), jnp.int32),
            pltpu.VMEM((_GK, _GCH, D), f32),
            pltpu.SemaphoreType.DMA,
            pltpu.SemaphoreType.DMA,
            pltpu.SemaphoreType.DMA,
        ],
    )
    def k(dbx_hbm, ext_hbm, src_hbm, dst_hbm, dbs_hbm, exd_hbm,
          sbuf, tbuf, dbuf, ebuf, sem_i, sem_g, sem_w):
        wid = lax.axis_index("s") * NCORE + lax.axis_index("c")
        base0 = wid * _EPW

        def fire_idx(r):
            for j in range(_GK):
                b = base0 + (r * _GK + j) * _GCH
                pltpu.async_copy(src_hbm.at[pl.ds(b, _GCH)], sbuf.at[j], sem_i)
                pltpu.async_copy(dst_hbm.at[pl.ds(b, _GCH)], tbuf.at[j], sem_i)

        def drain_idx():
            for j in range(_GK):
                pltpu.make_async_copy(src_hbm.at[pl.ds(base0, _GCH)], sbuf.at[j], sem_i).wait()
                pltpu.make_async_copy(dst_hbm.at[pl.ds(base0, _GCH)], tbuf.at[j], sem_i).wait()

        def fire_gathers():
            for j in range(_GK):
                pltpu.async_copy(dbx_hbm.at[sbuf.at[j]], dbuf.at[j], sem_g)
                pltpu.async_copy(ext_hbm.at[tbuf.at[j]], ebuf.at[j], sem_g)

        def drain_gathers():
            for j in range(_GK):
                pltpu.make_async_copy(dbx_hbm.at[sbuf.at[j]], dbuf.at[j], sem_g).wait()
                pltpu.make_async_copy(ext_hbm.at[tbuf.at[j]], ebuf.at[j], sem_g).wait()

        def fire_writes(r):
            for j in range(_GK):
                b = base0 + (r * _GK + j) * _GCH
                pltpu.async_copy(dbuf.at[j], dbs_hbm.at[pl.ds(b, _GCH)], sem_w)
                pltpu.async_copy(ebuf.at[j], exd_hbm.at[pl.ds(b, _GCH)], sem_w)

        def drain_writes():
            for j in range(_GK):
                pltpu.make_async_copy(dbuf.at[j], dbs_hbm.at[pl.ds(base0, _GCH)], sem_w).wait()
                pltpu.make_async_copy(ebuf.at[j], exd_hbm.at[pl.ds(base0, _GCH)], sem_w).wait()

        @pl.loop(0, _GROUNDS)
        def _(r):
            fire_idx(r)

            @pl.when(r > 0)
            def _():
                drain_writes()

            drain_idx()
            fire_gathers()
            drain_gathers()
            fire_writes(r)

        drain_writes()

    return k(dbx, ext, src, dst)


def _sc_scatter(comb, dst):
    comb = comb.reshape(2 * E, D)

    @functools.partial(
        pl.kernel,
        out_type=jax.ShapeDtypeStruct((2, _NPAD, D), f32),
        mesh=_sc_mesh(),
        scratch_types=[
            pltpu.VMEM((2, _SK, _SCH), jnp.int32),
            pltpu.VMEM((2, _SK, _SCH, D), f32),
            pltpu.VMEM((_ZR // 2, D), f32),
            pltpu.VMEM_SHARED((_NPAD, D), f32),
            pltpu.SemaphoreType.DMA,
            pltpu.SemaphoreType.DMA,
        ],
    )
    def k(comb_hbm, dst_hbm, acc_hbm, ibuf, mbuf, zbuf, acc_sh, sem_l0, sem_l1):
        cid = lax.axis_index("c")
        sid = lax.axis_index("s")
        row0 = sid * _RPT

        @pl.loop(0, _ZR // 2)
        def _(r):
            for j in range(D // 16):
                zbuf[r, pl.ds(j * 16, 16)] = jnp.zeros((16,), f32)

        @pl.loop(0, _RPT // (_ZR // 2))
        def _(t):
            pltpu.sync_copy(zbuf, acc_sh.at[pl.ds(row0 + t * (_ZR // 2), _ZR // 2)])

        plsc.subcore_barrier()

        base0 = sid * _EPT
        fbase0 = cid * E + base0
        sems = (sem_l0, sem_l1)

        def fire_group(r, sl):
            for j in range(_SK):
                off = (r * _SK + j) * _SCH
                pltpu.async_copy(dst_hbm.at[pl.ds(base0 + off, _SCH)],
                                 ibuf.at[sl, j], sems[sl])
                pltpu.async_copy(comb_hbm.at[pl.ds(fbase0 + off, _SCH)],
                                 mbuf.at[sl, j], sems[sl])

        def drain_group(sl):
            for j in range(_SK):
                pltpu.make_async_copy(dst_hbm.at[pl.ds(base0, _SCH)],
                                      ibuf.at[sl, j], sems[sl]).wait()
                pltpu.make_async_copy(comb_hbm.at[pl.ds(fbase0, _SCH)],
                                      mbuf.at[sl, j], sems[sl]).wait()

        fire_group(0, 0)

        @pl.loop(0, _SROUNDS - 1, step=2)
        def _(i):
            for sl in range(2):
                r = i + sl
                drain_group(sl)
                fire_group(r + 1, 1 - sl)
                for j in range(_SK):
                    pltpu.sync_copy(mbuf.at[sl, j], acc_sh.at[ibuf.at[sl, j]], add=True)

        drain_group(0)
        for j in range(_SK):
            pltpu.sync_copy(mbuf.at[0, j], acc_sh.at[ibuf.at[0, j]], add=True)

        plsc.subcore_barrier()

        @pl.loop(0, _RPT // (_ZR // 2))
        def _(t):
            r = row0 + t * (_ZR // 2)
            pltpu.sync_copy(acc_sh.at[pl.ds(r, _ZR // 2)], zbuf)
            pltpu.sync_copy(zbuf, acc_hbm.at[cid, pl.ds(r, _ZR // 2)])

    return k(comb, dst)


def kernel(x, edge_index, edge_attr, We, be, Ws, bs, Wh, bh):
    src = edge_index[0]
    dst = edge_index[1]
    ea = jnp.pad(edge_attr, ((0, 0), (0, 1)))
    wep = jnp.pad(We, ((0, 1), (0, 0)))

    e = None
    out = None
    for l in range(3):
        W = Ws[l]
        b = bs[l]
        wn = jnp.concatenate([W[0], W[3], W[1], W[4]], axis=1)
        bn = jnp.concatenate([b[0], b[3], b[1], b[4]])
        ax, dbx, ext = _node_proj(x, wn, bn)
        dbs, exd = _sc_gather(dbx, ext, src, dst)
        comb, e = _edge_elem(dbs, exd, e, ea, wep, be, W[2], b[2])
        acc2 = _sc_scatter(comb, dst)
        if l < 2:
            x = _x_update(x, ax, acc2)
        else:
            out = _x_update(x, ax, acc2, Wh, bh)
    return out
